# Initial kernel scaffold; baseline (speedup 1.0000x reference)
import jax
import jax.numpy as jnp
from jax.experimental import pallas as pl


def kernel(x, Wq, Wk, Wv, Wo, gq, gk, Wg, W1, b1, W2, b2):
    raise NotImplementedError("write your pallas kernel here")


# speedup vs baseline: 1.0794x; 1.0794x over previous
import jax
import jax.numpy as jnp
from jax.experimental import pallas as pl
from jax.experimental.pallas import tpu as pltpu

B, S, DIM = 1, 2048, 1024
DH, H = 64, 16
E, HID = 8, 4096
EPS = 1e-6
SCALE = DH ** -0.5

TQ = 256
TQA = 256
KH = 512
HB = HID // KH
GPAD = 128


def _qkv_body(x_ref, wq_ref, wk_ref, wv_ref, gq_ref, gk_ref, q_ref, k_ref, v_ref):
    xb = x_ref[...]
    q = jnp.dot(xb, wq_ref[...], preferred_element_type=jnp.float32)
    k = jnp.dot(xb, wk_ref[...], preferred_element_type=jnp.float32)
    v = jnp.dot(xb, wv_ref[...], preferred_element_type=jnp.float32)
    gq = gq_ref[...]
    gk = gk_ref[...]
    for hh in range(H):
        sl = slice(hh * DH, (hh + 1) * DH)
        qh = q[:, sl]
        kh = k[:, sl]
        qms = jnp.mean(qh * qh, axis=1, keepdims=True)
        kms = jnp.mean(kh * kh, axis=1, keepdims=True)
        q_ref[hh] = qh * jax.lax.rsqrt(qms + EPS) * gq[:, sl]
        k_ref[hh] = kh * jax.lax.rsqrt(kms + EPS) * gk[:, sl]
        v_ref[hh] = v[:, sl]


def _attn_body(q_ref, k_ref, v_ref, o_ref):
    qb = pl.program_id(1)
    q = q_ref[0]
    k = k_ref[0]
    s = jax.lax.dot_general(q, k, (((1,), (1,)), ((), ())),
                            preferred_element_type=jnp.float32) * SCALE
    rows = qb * TQA + jax.lax.broadcasted_iota(jnp.int32, (TQA, S), 0)
    cols = jax.lax.broadcasted_iota(jnp.int32, (TQA, S), 1)
    s = jnp.where(rows >= cols, s, -jnp.inf)
    m = jnp.max(s, axis=1, keepdims=True)
    p = jnp.exp(s - m)
    denom = jnp.sum(p, axis=1, keepdims=True)
    o = jnp.dot(p, v_ref[0], preferred_element_type=jnp.float32)
    o_ref[0] = o / denom


def _oproj_body(a_ref, wo_ref, wg_ref, o_ref, g_ref):
    ob = jnp.zeros((TQ, DIM), jnp.float32)
    for hh in range(H):
        ob = ob + jnp.dot(a_ref[hh], wo_ref[hh * DH:(hh + 1) * DH, :],
                          preferred_element_type=jnp.float32)
    o_ref[...] = ob
    gl = jnp.dot(ob, wg_ref[...], preferred_element_type=jnp.float32)
    cols = jax.lax.broadcasted_iota(jnp.int32, gl.shape, 1)
    gl = jnp.where(cols < E, gl, -jnp.inf)
    m = jnp.max(gl, axis=1, keepdims=True)
    p = jnp.exp(gl - m)
    g_ref[...] = p / jnp.sum(p, axis=1, keepdims=True)


def _moe_body(o_ref, g_ref, w1_ref, b1_ref, w2_ref, b2_ref, out_ref):
    e = pl.program_id(0)
    h = pl.program_id(1)

    @pl.when((e == 0) & (h == 0))
    def _init():
        out_ref[...] = jnp.zeros_like(out_ref)

    ob = o_ref[...]
    hb = jnp.dot(ob, w1_ref[0], preferred_element_type=jnp.float32) + b1_ref[0]
    hb = jax.nn.gelu(hb)
    contrib = jnp.dot(hb, w2_ref[0], preferred_element_type=jnp.float32)
    g = g_ref[...]
    cols = jax.lax.broadcasted_iota(jnp.int32, g.shape, 1)
    ge = jnp.sum(jnp.where(cols == e, g, 0.0), axis=1, keepdims=True)
    acc = out_ref[...] + ge * contrib

    @pl.when(h == HB - 1)
    def _bias():
        out_ref[...] = acc + ge * b2_ref[0]

    @pl.when(h != HB - 1)
    def _noacc():
        out_ref[...] = acc


def kernel(x, Wq, Wk, Wv, Wo, gq, gk, Wg, W1, b1, W2, b2):
    xs = x.reshape(S, DIM)
    gq_t = jnp.tile(gq, H).reshape(1, H * DH)
    gk_t = jnp.tile(gk, H).reshape(1, H * DH)
    wg_pad = jnp.zeros((DIM, GPAD), Wg.dtype).at[:, :E].set(Wg)
    b1_3d = b1.reshape(E, 1, HID)
    b2_3d = b2.reshape(E, 1, DIM)

    q, k, v = pl.pallas_call(
        _qkv_body,
        grid=(S // TQ,),
        in_specs=[
            pl.BlockSpec((TQ, DIM), lambda i: (i, 0)),
            pl.BlockSpec((DIM, H * DH), lambda i: (0, 0)),
            pl.BlockSpec((DIM, H * DH), lambda i: (0, 0)),
            pl.BlockSpec((DIM, H * DH), lambda i: (0, 0)),
            pl.BlockSpec((1, H * DH), lambda i: (0, 0)),
            pl.BlockSpec((1, H * DH), lambda i: (0, 0)),
        ],
        out_specs=[
            pl.BlockSpec((H, TQ, DH), lambda i: (0, i, 0)),
            pl.BlockSpec((H, TQ, DH), lambda i: (0, i, 0)),
            pl.BlockSpec((H, TQ, DH), lambda i: (0, i, 0)),
        ],
        out_shape=[jax.ShapeDtypeStruct((H, S, DH), jnp.float32)] * 3,
    )(xs, Wq, Wk, Wv, gq_t, gk_t)

    attn = pl.pallas_call(
        _attn_body,
        grid=(H, S // TQA),
        in_specs=[
            pl.BlockSpec((1, TQA, DH), lambda hh, i: (hh, i, 0)),
            pl.BlockSpec((1, S, DH), lambda hh, i: (hh, 0, 0)),
            pl.BlockSpec((1, S, DH), lambda hh, i: (hh, 0, 0)),
        ],
        out_specs=pl.BlockSpec((1, TQA, DH), lambda hh, i: (hh, i, 0)),
        out_shape=jax.ShapeDtypeStruct((H, S, DH), jnp.float32),
    )(q, k, v)

    o, gate = pl.pallas_call(
        _oproj_body,
        grid=(S // TQ,),
        in_specs=[
            pl.BlockSpec((H, TQ, DH), lambda i: (0, i, 0)),
            pl.BlockSpec((H * DH, DIM), lambda i: (0, 0)),
            pl.BlockSpec((DIM, GPAD), lambda i: (0, 0)),
        ],
        out_specs=[
            pl.BlockSpec((TQ, DIM), lambda i: (i, 0)),
            pl.BlockSpec((TQ, GPAD), lambda i: (i, 0)),
        ],
        out_shape=[
            jax.ShapeDtypeStruct((S, DIM), jnp.float32),
            jax.ShapeDtypeStruct((S, GPAD), jnp.float32),
        ],
    )(attn, Wo, wg_pad)

    out = pl.pallas_call(
        _moe_body,
        grid=(E, HB),
        in_specs=[
            pl.BlockSpec((S, DIM), lambda e, hh: (0, 0)),
            pl.BlockSpec((S, GPAD), lambda e, hh: (0, 0)),
            pl.BlockSpec((1, DIM, KH), lambda e, hh: (e, 0, hh)),
            pl.BlockSpec((1, 1, KH), lambda e, hh: (e, 0, hh)),
            pl.BlockSpec((1, KH, DIM), lambda e, hh: (e, hh, 0)),
            pl.BlockSpec((1, 1, DIM), lambda e, hh: (e, 0, 0)),
        ],
        out_specs=pl.BlockSpec((S, DIM), lambda e, hh: (0, 0)),
        out_shape=jax.ShapeDtypeStruct((S, DIM), jnp.float32),
    )(o, gate, W1, b1_3d, W2, b2_3d)

    return out.reshape(B, S, DIM)


# speedup vs baseline: 1.1497x; 1.0651x over previous
import jax
import jax.numpy as jnp
from jax.experimental import pallas as pl
from jax.experimental.pallas import tpu as pltpu

B, S, DIM = 1, 2048, 1024
DH, H = 64, 16
E, HID = 8, 4096
EPS = 1e-6
SCALE = DH ** -0.5

TQ = 256
TQA = 256
KH = 512
HB = HID // KH
GPAD = 128

F32 = jnp.float32
BF16 = jnp.bfloat16


def _bdot(a, b):
    return jax.lax.dot_general(a.astype(BF16), b.astype(BF16),
                               (((1,), (0,)), ((), ())),
                               preferred_element_type=F32)


def _qkv_body(x_ref, wq_ref, wk_ref, wv_ref, gq_ref, gk_ref, q_ref, k_ref, v_ref):
    xb = x_ref[...].astype(BF16)
    q = jnp.dot(xb, wq_ref[...], preferred_element_type=F32)
    k = jnp.dot(xb, wk_ref[...], preferred_element_type=F32)
    v = jnp.dot(xb, wv_ref[...], preferred_element_type=F32)
    gq = gq_ref[...]
    gk = gk_ref[...]
    for hh in range(H):
        sl = slice(hh * DH, (hh + 1) * DH)
        qh = q[:, sl]
        kh = k[:, sl]
        qms = jnp.mean(qh * qh, axis=1, keepdims=True)
        kms = jnp.mean(kh * kh, axis=1, keepdims=True)
        q_ref[hh] = (qh * jax.lax.rsqrt(qms + EPS) * gq[:, sl]).astype(BF16)
        k_ref[hh] = (kh * jax.lax.rsqrt(kms + EPS) * gk[:, sl]).astype(BF16)
        v_ref[hh] = v[:, sl].astype(BF16)


def _attn_body(q_ref, k_ref, v_ref, o_ref):
    qb = pl.program_id(1)
    q = q_ref[0]
    k = k_ref[0]
    s = jax.lax.dot_general(q, k, (((1,), (1,)), ((), ())),
                            preferred_element_type=F32) * SCALE
    rows = qb * TQA + jax.lax.broadcasted_iota(jnp.int32, (TQA, S), 0)
    cols = jax.lax.broadcasted_iota(jnp.int32, (TQA, S), 1)
    s = jnp.where(rows >= cols, s, -jnp.inf)
    m = jnp.max(s, axis=1, keepdims=True)
    p = jnp.exp(s - m)
    denom = jnp.sum(p, axis=1, keepdims=True)
    o = jnp.dot(p.astype(BF16), v_ref[0], preferred_element_type=F32)
    o_ref[0] = (o / denom).astype(BF16)


def _oproj_body(a_ref, wo_ref, wg_ref, o_ref, g_ref):
    ob = jnp.zeros((TQ, DIM), F32)
    for hh in range(H):
        ob = ob + _bdot(a_ref[hh], wo_ref[hh * DH:(hh + 1) * DH, :])
    o_ref[...] = ob.astype(BF16)
    gl = _bdot(ob, wg_ref[...])
    cols = jax.lax.broadcasted_iota(jnp.int32, gl.shape, 1)
    gl = jnp.where(cols < E, gl, -jnp.inf)
    m = jnp.max(gl, axis=1, keepdims=True)
    p = jnp.exp(gl - m)
    g_ref[...] = p / jnp.sum(p, axis=1, keepdims=True)


def _moe_body(o_ref, g_ref, w1_ref, b1_ref, w2_ref, b2_ref, out_ref):
    e = pl.program_id(0)
    h = pl.program_id(1)

    @pl.when((e == 0) & (h == 0))
    def _init():
        out_ref[...] = jnp.zeros_like(out_ref)

    ob = o_ref[...]
    hb = _bdot(ob, w1_ref[0]) + b1_ref[0]
    hb = jax.nn.gelu(hb)
    contrib = _bdot(hb, w2_ref[0])
    g = g_ref[...]
    cols = jax.lax.broadcasted_iota(jnp.int32, g.shape, 1)
    ge = jnp.sum(jnp.where(cols == e, g, 0.0), axis=1, keepdims=True)
    acc = out_ref[...] + ge * contrib

    @pl.when(h == HB - 1)
    def _bias():
        out_ref[...] = acc + ge * b2_ref[0]

    @pl.when(h != HB - 1)
    def _noacc():
        out_ref[...] = acc


def kernel(x, Wq, Wk, Wv, Wo, gq, gk, Wg, W1, b1, W2, b2):
    xs = x.reshape(S, DIM)
    gq_t = jnp.tile(gq, H).reshape(1, H * DH)
    gk_t = jnp.tile(gk, H).reshape(1, H * DH)
    wg_pad = jnp.zeros((DIM, GPAD), Wg.dtype).at[:, :E].set(Wg)
    b1_3d = b1.reshape(E, 1, HID)
    b2_3d = b2.reshape(E, 1, DIM)
    wq_b, wk_b, wv_b = Wq.astype(BF16), Wk.astype(BF16), Wv.astype(BF16)

    q, k, v = pl.pallas_call(
        _qkv_body,
        grid=(S // TQ,),
        in_specs=[
            pl.BlockSpec((TQ, DIM), lambda i: (i, 0)),
            pl.BlockSpec((DIM, H * DH), lambda i: (0, 0)),
            pl.BlockSpec((DIM, H * DH), lambda i: (0, 0)),
            pl.BlockSpec((DIM, H * DH), lambda i: (0, 0)),
            pl.BlockSpec((1, H * DH), lambda i: (0, 0)),
            pl.BlockSpec((1, H * DH), lambda i: (0, 0)),
        ],
        out_specs=[
            pl.BlockSpec((H, TQ, DH), lambda i: (0, i, 0)),
            pl.BlockSpec((H, TQ, DH), lambda i: (0, i, 0)),
            pl.BlockSpec((H, TQ, DH), lambda i: (0, i, 0)),
        ],
        out_shape=[jax.ShapeDtypeStruct((H, S, DH), BF16)] * 3,
    )(xs, wq_b, wk_b, wv_b, gq_t, gk_t)

    attn = pl.pallas_call(
        _attn_body,
        grid=(H, S // TQA),
        in_specs=[
            pl.BlockSpec((1, TQA, DH), lambda hh, i: (hh, i, 0)),
            pl.BlockSpec((1, S, DH), lambda hh, i: (hh, 0, 0)),
            pl.BlockSpec((1, S, DH), lambda hh, i: (hh, 0, 0)),
        ],
        out_specs=pl.BlockSpec((1, TQA, DH), lambda hh, i: (hh, i, 0)),
        out_shape=jax.ShapeDtypeStruct((H, S, DH), BF16),
    )(q, k, v)

    o, gate = pl.pallas_call(
        _oproj_body,
        grid=(S // TQ,),
        in_specs=[
            pl.BlockSpec((H, TQ, DH), lambda i: (0, i, 0)),
            pl.BlockSpec((H * DH, DIM), lambda i: (0, 0)),
            pl.BlockSpec((DIM, GPAD), lambda i: (0, 0)),
        ],
        out_specs=[
            pl.BlockSpec((TQ, DIM), lambda i: (i, 0)),
            pl.BlockSpec((TQ, GPAD), lambda i: (i, 0)),
        ],
        out_shape=[
            jax.ShapeDtypeStruct((S, DIM), BF16),
            jax.ShapeDtypeStruct((S, GPAD), F32),
        ],
    )(attn, Wo, wg_pad)

    out = pl.pallas_call(
        _moe_body,
        grid=(E, HB),
        in_specs=[
            pl.BlockSpec((S, DIM), lambda e, hh: (0, 0)),
            pl.BlockSpec((S, GPAD), lambda e, hh: (0, 0)),
            pl.BlockSpec((1, DIM, KH), lambda e, hh: (e, 0, hh)),
            pl.BlockSpec((1, 1, KH), lambda e, hh: (e, 0, hh)),
            pl.BlockSpec((1, KH, DIM), lambda e, hh: (e, hh, 0)),
            pl.BlockSpec((1, 1, DIM), lambda e, hh: (e, 0, 0)),
        ],
        out_specs=pl.BlockSpec((S, DIM), lambda e, hh: (0, 0)),
        out_shape=jax.ShapeDtypeStruct((S, DIM), F32),
    )(o, gate, W1, b1_3d, W2, b2_3d)

    return out.reshape(B, S, DIM)
